# Initial kernel scaffold; baseline (speedup 1.0000x reference)
import jax
import jax.numpy as jnp
from jax.experimental import pallas as pl


def kernel(node_features, edge_index, edge_features, batch, params):
    raise NotImplementedError("write your pallas kernel here")


# speedup vs baseline: 2.2547x; 2.2547x over previous
import functools

import jax
import jax.numpy as jnp
from jax import lax
from jax.experimental import pallas as pl
from jax.experimental.pallas import tpu as pltpu
from jax.experimental.pallas import tpu_sc as plsc

N = 50000
E = 800000
ED = 2 * E
NG = 64

NPAD = 51200
TILES = 16
ROWS_PER_TILE = NPAD // TILES
ZCOPIES = ROWS_PER_TILE // 128
CHUNK = 128
EPT = ED // TILES
NCH = -(-EPT // CHUNK)
DEG_NCH = NCH // 2

RB = 2000
NBLK = N // RB
EB = 4000
NEBLK = E // EB

_F32 = jnp.float32


def _node_mlp_body(nf, nW1, nb1, nW2, nb2, mW1x, x_out, xh0_out, xh1_out):
    a = jnp.maximum(jnp.dot(nf[...], nW1[...], preferred_element_type=_F32)
                    + nb1[...], 0.0)
    x = jnp.maximum(jnp.dot(a, nW2[...], preferred_element_type=_F32)
                    + nb2[...], 0.0)
    xh = jnp.dot(x, mW1x[...], preferred_element_type=_F32)
    x_out[...] = x
    xh0_out[...] = xh[:, :32]
    xh1_out[...] = xh[:, 32:]


def _node_mlp(nf, nW1, nb1, nW2, nb2, mW1x):
    full = lambda s: pl.BlockSpec(s, lambda i: (0, 0))
    row = lambda c: pl.BlockSpec((RB, c), lambda i: (i, 0))
    return pl.pallas_call(
        _node_mlp_body,
        grid=(NBLK,),
        in_specs=[row(128), full((128, 128)), full((1, 128)),
                  full((128, 64)), full((1, 64)), full((64, 64))],
        out_specs=[row(64), row(32), row(32)],
        out_shape=[jax.ShapeDtypeStruct((N, 64), _F32),
                   jax.ShapeDtypeStruct((N, 32), _F32),
                   jax.ShapeDtypeStruct((N, 32), _F32)],
    )(nf, nW1, nb1, nW2, nb2, mW1x)


def _edge_mlp_body(ef, eW1, eb1, eW2, eb2, m1e, m1b, m2e, m2b,
                   o10, o11, o20, o21):
    a = jnp.maximum(jnp.dot(ef[...], eW1[...], preferred_element_type=_F32)
                    + eb1[...], 0.0)
    ea = jnp.maximum(jnp.dot(a, eW2[...], preferred_element_type=_F32)
                     + eb2[...], 0.0)
    eh1 = jnp.dot(ea, m1e[...], preferred_element_type=_F32) + m1b[...]
    eh2 = jnp.dot(ea, m2e[...], preferred_element_type=_F32) + m2b[...]
    o10[...] = eh1[:, :32]
    o11[...] = eh1[:, 32:]
    o20[...] = eh2[:, :32]
    o21[...] = eh2[:, 32:]


def _edge_mlp(ef, eW1, eb1, eW2, eb2, m1e, m1b, m2e, m2b):
    full = lambda s: pl.BlockSpec(s, lambda i: (0, 0))
    row = lambda c: pl.BlockSpec((EB, c), lambda i: (i, 0))
    return pl.pallas_call(
        _edge_mlp_body,
        grid=(NEBLK,),
        in_specs=[row(16), full((16, 32)), full((1, 32)),
                  full((32, 16)), full((1, 16)),
                  full((16, 64)), full((1, 64)),
                  full((16, 64)), full((1, 64))],
        out_specs=[row(32), row(32), row(32), row(32)],
        out_shape=[jax.ShapeDtypeStruct((E, 32), _F32)] * 4,
    )(ef, eW1, eb1, eW2, eb2, m1e, m1b, m2e, m2b)


def _make_update_body(with_xh):
    def body(a0, a1, d0, d1, x, mW2, mb2, uWa, uWx, ub, *rest):
        if with_xh:
            (nW1x, xn_out, xh0_out, xh1_out) = rest
        else:
            (xn_out,) = rest
        s = jnp.concatenate([a0[...], a1[...]], axis=1)
        deg = d0[:, :1] + d1[:, :1]
        aggr = jnp.dot(s, mW2[...], preferred_element_type=_F32) + deg * mb2[...]
        xn = jnp.maximum(jnp.dot(aggr, uWa[...], preferred_element_type=_F32)
                         + jnp.dot(x[...], uWx[...], preferred_element_type=_F32)
                         + ub[...], 0.0)
        xn_out[...] = xn
        if with_xh:
            xh = jnp.dot(xn, nW1x[...], preferred_element_type=_F32)
            xh0_out[...] = xh[:, :32]
            xh1_out[...] = xh[:, 32:]
    return body


def _update(a0, a1, d0, d1, x, mW2, mb2, uWa, uWx, ub, nW1x=None):
    with_xh = nW1x is not None
    full = lambda s: pl.BlockSpec(s, lambda i: (0, 0))
    row = lambda c: pl.BlockSpec((RB, c), lambda i: (i, 0))
    in_specs = [row(32), row(32), row(16), row(16), row(64),
                full((64, 64)), full((1, 64)), full((64, 64)),
                full((64, 64)), full((1, 64))]
    args = [a0, a1, d0, d1, x, mW2, mb2, uWa, uWx, ub]
    out_specs = [row(64)]
    out_shape = [jax.ShapeDtypeStruct((N, 64), _F32)]
    if with_xh:
        in_specs.append(full((64, 64)))
        args.append(nW1x)
        out_specs += [row(32), row(32)]
        out_shape += [jax.ShapeDtypeStruct((N, 32), _F32)] * 2
    res = pl.pallas_call(
        _make_update_body(with_xh),
        grid=(NBLK,),
        in_specs=in_specs,
        out_specs=out_specs,
        out_shape=out_shape,
    )(*args)
    return res if with_xh else res[0]


def _readout_body(x, b, gateW, gateb, outW, outb, out, gmax_s, num_s):
    ph = pl.program_id(0)
    i = pl.program_id(1)

    @pl.when((ph == 0) & (i == 0))
    def _init():
        gmax_s[...] = jnp.full((1, NG), -1e30, _F32)
        num_s[...] = jnp.zeros((NG, 72), _F32)

    bcol = b[0]
    seg = lax.broadcasted_iota(jnp.int32, (1, NG), 1)
    mask = bcol == seg
    gate = jnp.dot(x[...], gateW[...], preferred_element_type=_F32) + gateb[...]

    @pl.when(ph == 0)
    def _maxpass():
        contrib = jnp.where(mask, gate, -1e30)
        gmax_s[...] = jnp.maximum(gmax_s[...],
                                  jnp.max(contrib, axis=0, keepdims=True))

    @pl.when(ph == 1)
    def _sumpass():
        gm = gmax_s[...]
        gm = jnp.where(gm > -1e29, gm, 0.0)
        gsel = jnp.sum(jnp.where(mask, gm, 0.0), axis=1, keepdims=True)
        e = jnp.exp(gate - gsel)
        em = jnp.where(mask, e, 0.0)
        xext = jnp.concatenate([x[...], jnp.ones((RB, 8), _F32)], axis=1)
        num_s[...] += lax.dot_general(em, xext,
                                      (((0,), (0,)), ((), ())),
                                      preferred_element_type=_F32)

    @pl.when((ph == 2) & (i == 0))
    def _finish():
        nv = num_s[...]
        den = nv[:, 64:65]
        ro = nv[:, :64] / (den + 1e-16)
        out[...] = jnp.dot(ro, outW[...], preferred_element_type=_F32) + outb[...]


def _readout(x, batch3, gateW, gateb, outW, outb):
    full = lambda s: pl.BlockSpec(s, lambda ph, i: (0, 0))
    return pl.pallas_call(
        _readout_body,
        grid=(3, NBLK),
        in_specs=[pl.BlockSpec((RB, 64), lambda ph, i: (i, 0)),
                  pl.BlockSpec((1, RB, 1), lambda ph, i: (i, 0, 0)),
                  full((64, 1)), full((1, 1)), full((64, 32)), full((1, 32))],
        out_specs=pl.BlockSpec((NG, 32), lambda ph, i: (0, 0)),
        out_shape=jax.ShapeDtypeStruct((NG, 32), _F32),
        scratch_shapes=[pltpu.VMEM((1, NG), _F32), pltpu.VMEM((NG, 72), _F32)],
    )(x, batch3, gateW, gateb, outW, outb)


_MESH = plsc.VectorSubcoreMesh(core_axis_name="c", subcore_axis_name="s",
                               num_cores=2, num_subcores=16)


def _edge_pass_body(xh0, xh1, eh0, eh1, sidx, eidx, didx, out0, out1,
                    sbuf, ebuf, dbuf, xrows, erows, zbuf, aggr, sem):
    cid = lax.axis_index("c")
    sid = lax.axis_index("s")
    base = sid * ROWS_PER_TILE

    def zrow(i, _):
        zbuf[i, pl.ds(0, 16)] = jnp.zeros((16,), _F32)
        zbuf[i, pl.ds(16, 16)] = jnp.zeros((16,), _F32)
        return _
    lax.fori_loop(0, CHUNK, zrow, None)

    def zcp(c, _):
        pltpu.sync_copy(zbuf, aggr.at[pl.ds(base + c * CHUNK, CHUNK)])
        return _
    lax.fori_loop(0, ZCOPIES, zcp, None)
    plsc.subcore_barrier()

    def chunk(c, _):
        pltpu.sync_copy(sidx.at[sid, c], sbuf)
        pltpu.sync_copy(eidx.at[sid, c], ebuf)
        pltpu.sync_copy(didx.at[sid, c], dbuf)

        @pl.when(cid == 0)
        def _g0():
            pltpu.async_copy(xh0.at[sbuf], xrows, sem).wait()
            pltpu.async_copy(eh0.at[ebuf], erows, sem).wait()

        @pl.when(cid == 1)
        def _g1():
            pltpu.async_copy(xh1.at[sbuf], xrows, sem).wait()
            pltpu.async_copy(eh1.at[ebuf], erows, sem).wait()

        def rows(i, _):
            s0 = pl.ds(0, 16)
            s1 = pl.ds(16, 16)
            xrows[i, s0] = jnp.maximum(xrows[i, s0] + erows[i, s0], 0.0)
            xrows[i, s1] = jnp.maximum(xrows[i, s1] + erows[i, s1], 0.0)
            return _
        lax.fori_loop(0, CHUNK, rows, None)

        pltpu.sync_copy(xrows, aggr.at[dbuf], add=True)
        return _
    lax.fori_loop(0, NCH, chunk, None)
    plsc.subcore_barrier()

    @pl.when(cid == 0)
    def _w0():
        pltpu.sync_copy(aggr.at[pl.ds(base, ROWS_PER_TILE)],
                        out0.at[pl.ds(base, ROWS_PER_TILE)])

    @pl.when(cid == 1)
    def _w1():
        pltpu.sync_copy(aggr.at[pl.ds(base, ROWS_PER_TILE)],
                        out1.at[pl.ds(base, ROWS_PER_TILE)])


_edge_pass = pl.kernel(
    _edge_pass_body,
    out_type=[jax.ShapeDtypeStruct((NPAD, 32), _F32)] * 2,
    mesh=_MESH,
    compiler_params=pltpu.CompilerParams(use_tc_tiling_on_sc=False),
    scratch_types=[
        pltpu.VMEM((CHUNK,), jnp.int32),
        pltpu.VMEM((CHUNK,), jnp.int32),
        pltpu.VMEM((CHUNK,), jnp.int32),
        pltpu.VMEM((CHUNK, 32), _F32),
        pltpu.VMEM((CHUNK, 32), _F32),
        pltpu.VMEM((CHUNK, 32), _F32),
        pltpu.VMEM_SHARED((NPAD, 32), _F32),
        pltpu.SemaphoreType.DMA,
    ],
)


def _deg_body(didx, out0, out1, dbuf, ones, zbuf, degs):
    cid = lax.axis_index("c")
    sid = lax.axis_index("s")
    base = sid * ROWS_PER_TILE

    def fill(i, _):
        ones[i, pl.ds(0, 16)] = jnp.ones((16,), _F32)
        zbuf[i, pl.ds(0, 16)] = jnp.zeros((16,), _F32)
        return _
    lax.fori_loop(0, CHUNK, fill, None)

    def zcp(c, _):
        pltpu.sync_copy(zbuf, degs.at[pl.ds(base + c * CHUNK, CHUNK)])
        return _
    lax.fori_loop(0, ZCOPIES, zcp, None)
    plsc.subcore_barrier()

    def chunk(c, _):
        pltpu.sync_copy(didx.at[sid, cid * DEG_NCH + c], dbuf)
        pltpu.sync_copy(ones, degs.at[dbuf], add=True)
        return _
    lax.fori_loop(0, DEG_NCH, chunk, None)
    plsc.subcore_barrier()

    @pl.when(cid == 0)
    def _w0():
        pltpu.sync_copy(degs.at[pl.ds(base, ROWS_PER_TILE)],
                        out0.at[pl.ds(base, ROWS_PER_TILE)])

    @pl.when(cid == 1)
    def _w1():
        pltpu.sync_copy(degs.at[pl.ds(base, ROWS_PER_TILE)],
                        out1.at[pl.ds(base, ROWS_PER_TILE)])


_deg = pl.kernel(
    _deg_body,
    out_type=[jax.ShapeDtypeStruct((NPAD, 16), _F32)] * 2,
    mesh=_MESH,
    compiler_params=pltpu.CompilerParams(use_tc_tiling_on_sc=False),
    scratch_types=[
        pltpu.VMEM((CHUNK,), jnp.int32),
        pltpu.VMEM((CHUNK, 16), _F32),
        pltpu.VMEM((CHUNK, 16), _F32),
        pltpu.VMEM_SHARED((NPAD, 16), _F32),
    ],
)


def _tile_pack(a, pad_val):
    a = a.reshape(TILES, EPT)
    a = jnp.pad(a, ((0, 0), (0, NCH * CHUNK - EPT)), constant_values=pad_val)
    return a.reshape(TILES, NCH, CHUNK)


def kernel(node_features, edge_index, edge_features, batch, params):
    p = params
    r1 = lambda v: v.reshape(1, -1)

    x, xh0, xh1 = _node_mlp(node_features, p['nW1'], r1(p['nb1']),
                            p['nW2'], r1(p['nb2']), p['g1mW1'][:64])
    eh10, eh11, eh20, eh21 = _edge_mlp(
        edge_features, p['eW1'], r1(p['eb1']), p['eW2'], r1(p['eb2']),
        p['g1mW1'][64:], r1(p['g1mb1']), p['g2mW1'][64:], r1(p['g2mb1']))

    srcd = jnp.concatenate([edge_index[0], edge_index[1]])
    dstd = jnp.concatenate([edge_index[1], edge_index[0]])
    eidd = jnp.concatenate([jnp.arange(E, dtype=jnp.int32)] * 2)
    sidx = _tile_pack(srcd, 0)
    didx = _tile_pack(dstd, N)
    eidx = _tile_pack(eidd, 0)

    deg0, deg1 = _deg(didx)

    a0, a1 = _edge_pass(xh0, xh1, eh10, eh11, sidx, eidx, didx)
    x, xh0, xh1 = _update(a0, a1, deg0, deg1, x, p['g1mW2'], r1(p['g1mb2']),
                          p['g1uW'][:64], p['g1uW'][64:], r1(p['g1ub']),
                          nW1x=p['g2mW1'][:64])

    a0, a1 = _edge_pass(xh0, xh1, eh20, eh21, sidx, eidx, didx)
    x = _update(a0, a1, deg0, deg1, x, p['g2mW2'], r1(p['g2mb2']),
                p['g2uW'][:64], p['g2uW'][64:], r1(p['g2ub']))

    batch3 = batch.reshape(NBLK, RB, 1)
    return _readout(x, batch3, p['gateW'], r1(p['gateb']),
                    p['outW'], r1(p['outb']))


# speedup vs baseline: 4.8540x; 2.1528x over previous
import functools

import jax
import jax.numpy as jnp
from jax import lax
from jax.experimental import pallas as pl
from jax.experimental.pallas import tpu as pltpu
from jax.experimental.pallas import tpu_sc as plsc

N = 50000
E = 800000
ED = 2 * E
NG = 64

NPAD = 51200
TILES = 16
ROWS_PER_TILE = NPAD // TILES
ZCOPIES = ROWS_PER_TILE // 128
CHUNK = 128
EPT = ED // TILES
NCH = -(-EPT // CHUNK)
DEG_NCH = NCH // 2
GCH = 34
NGROUPS = NCH // GCH
PAIRS = GCH // 2
E_PAD = EPT + (NCH * CHUNK - EPT) + 7 * EPT

RB = 2000
NBLK = N // RB
EB = 4000
NEBLK = E // EB

_F32 = jnp.float32


def _node_mlp_body(nf, nW1, nb1, nW2, nb2, mW1x, x_out, xh0_out, xh1_out):
    a = jnp.maximum(jnp.dot(nf[...], nW1[...], preferred_element_type=_F32)
                    + nb1[...], 0.0)
    x = jnp.maximum(jnp.dot(a, nW2[...], preferred_element_type=_F32)
                    + nb2[...], 0.0)
    xh = jnp.dot(x, mW1x[...], preferred_element_type=_F32)
    x_out[...] = x
    xh0_out[...] = xh[:, :32]
    xh1_out[...] = xh[:, 32:]


def _node_mlp(nf, nW1, nb1, nW2, nb2, mW1x):
    full = lambda s: pl.BlockSpec(s, lambda i: (0, 0))
    row = lambda c: pl.BlockSpec((RB, c), lambda i: (i, 0))
    return pl.pallas_call(
        _node_mlp_body,
        grid=(NBLK,),
        in_specs=[row(128), full((128, 128)), full((1, 128)),
                  full((128, 64)), full((1, 64)), full((64, 64))],
        out_specs=[row(64), row(32), row(32)],
        out_shape=[jax.ShapeDtypeStruct((N, 64), _F32),
                   jax.ShapeDtypeStruct((N, 32), _F32),
                   jax.ShapeDtypeStruct((N, 32), _F32)],
    )(nf, nW1, nb1, nW2, nb2, mW1x)


def _edge_mlp_body(ef, eW1, eb1, eW2, eb2, m1e, m1b, m2e, m2b,
                   o10, o11, o20, o21):
    a = jnp.maximum(jnp.dot(ef[...], eW1[...], preferred_element_type=_F32)
                    + eb1[...], 0.0)
    ea = jnp.maximum(jnp.dot(a, eW2[...], preferred_element_type=_F32)
                     + eb2[...], 0.0)
    eh1 = jnp.dot(ea, m1e[...], preferred_element_type=_F32) + m1b[...]
    eh2 = jnp.dot(ea, m2e[...], preferred_element_type=_F32) + m2b[...]
    o10[...] = eh1[:, :32]
    o11[...] = eh1[:, 32:]
    o20[...] = eh2[:, :32]
    o21[...] = eh2[:, 32:]


def _edge_mlp(ef, eW1, eb1, eW2, eb2, m1e, m1b, m2e, m2b):
    full = lambda s: pl.BlockSpec(s, lambda i: (0, 0))
    row = lambda c: pl.BlockSpec((EB, c), lambda i: (i, 0))
    return pl.pallas_call(
        _edge_mlp_body,
        grid=(NEBLK,),
        in_specs=[row(16), full((16, 32)), full((1, 32)),
                  full((32, 16)), full((1, 16)),
                  full((16, 64)), full((1, 64)),
                  full((16, 64)), full((1, 64))],
        out_specs=[row(32), row(32), row(32), row(32)],
        out_shape=[jax.ShapeDtypeStruct((E_PAD, 32), _F32)] * 4,
    )(ef, eW1, eb1, eW2, eb2, m1e, m1b, m2e, m2b)


def _make_update_body(with_xh):
    def body(a0, a1, d0, d1, x, mW2, mb2, uWa, uWx, ub, *rest):
        if with_xh:
            (nW1x, xn_out, xh0_out, xh1_out) = rest
        else:
            (xn_out,) = rest
        s = jnp.concatenate([a0[...], a1[...]], axis=1)
        deg = d0[:, :1] + d1[:, :1]
        aggr = jnp.dot(s, mW2[...], preferred_element_type=_F32) + deg * mb2[...]
        xn = jnp.maximum(jnp.dot(aggr, uWa[...], preferred_element_type=_F32)
                         + jnp.dot(x[...], uWx[...], preferred_element_type=_F32)
                         + ub[...], 0.0)
        xn_out[...] = xn
        if with_xh:
            xh = jnp.dot(xn, nW1x[...], preferred_element_type=_F32)
            xh0_out[...] = xh[:, :32]
            xh1_out[...] = xh[:, 32:]
    return body


def _update(a0, a1, d0, d1, x, mW2, mb2, uWa, uWx, ub, nW1x=None):
    with_xh = nW1x is not None
    full = lambda s: pl.BlockSpec(s, lambda i: (0, 0))
    row = lambda c: pl.BlockSpec((RB, c), lambda i: (i, 0))
    in_specs = [row(32), row(32), row(16), row(16), row(64),
                full((64, 64)), full((1, 64)), full((64, 64)),
                full((64, 64)), full((1, 64))]
    args = [a0, a1, d0, d1, x, mW2, mb2, uWa, uWx, ub]
    out_specs = [row(64)]
    out_shape = [jax.ShapeDtypeStruct((N, 64), _F32)]
    if with_xh:
        in_specs.append(full((64, 64)))
        args.append(nW1x)
        out_specs += [row(32), row(32)]
        out_shape += [jax.ShapeDtypeStruct((N, 32), _F32)] * 2
    res = pl.pallas_call(
        _make_update_body(with_xh),
        grid=(NBLK,),
        in_specs=in_specs,
        out_specs=out_specs,
        out_shape=out_shape,
    )(*args)
    return res if with_xh else res[0]


def _readout_body(x, b, gateW, gateb, outW, outb, out, gmax_s, num_s):
    ph = pl.program_id(0)
    i = pl.program_id(1)

    @pl.when((ph == 0) & (i == 0))
    def _init():
        gmax_s[...] = jnp.full((1, NG), -1e30, _F32)
        num_s[...] = jnp.zeros((NG, 72), _F32)

    bcol = b[0]
    seg = lax.broadcasted_iota(jnp.int32, (1, NG), 1)
    mask = bcol == seg
    gate = jnp.dot(x[...], gateW[...], preferred_element_type=_F32) + gateb[...]

    @pl.when(ph == 0)
    def _maxpass():
        contrib = jnp.where(mask, gate, -1e30)
        gmax_s[...] = jnp.maximum(gmax_s[...],
                                  jnp.max(contrib, axis=0, keepdims=True))

    @pl.when(ph == 1)
    def _sumpass():
        gm = gmax_s[...]
        gm = jnp.where(gm > -1e29, gm, 0.0)
        gsel = jnp.sum(jnp.where(mask, gm, 0.0), axis=1, keepdims=True)
        e = jnp.exp(gate - gsel)
        em = jnp.where(mask, e, 0.0)
        xext = jnp.concatenate([x[...], jnp.ones((RB, 8), _F32)], axis=1)
        num_s[...] += lax.dot_general(em, xext,
                                      (((0,), (0,)), ((), ())),
                                      preferred_element_type=_F32)

    @pl.when((ph == 2) & (i == 0))
    def _finish():
        nv = num_s[...]
        den = nv[:, 64:65]
        ro = nv[:, :64] / (den + 1e-16)
        out[...] = jnp.dot(ro, outW[...], preferred_element_type=_F32) + outb[...]


def _readout(x, batch3, gateW, gateb, outW, outb):
    full = lambda s: pl.BlockSpec(s, lambda ph, i: (0, 0))
    return pl.pallas_call(
        _readout_body,
        grid=(3, NBLK),
        in_specs=[pl.BlockSpec((RB, 64), lambda ph, i: (i, 0)),
                  pl.BlockSpec((1, RB, 1), lambda ph, i: (i, 0, 0)),
                  full((64, 1)), full((1, 1)), full((64, 32)), full((1, 32))],
        out_specs=pl.BlockSpec((NG, 32), lambda ph, i: (0, 0)),
        out_shape=jax.ShapeDtypeStruct((NG, 32), _F32),
        scratch_shapes=[pltpu.VMEM((1, NG), _F32), pltpu.VMEM((NG, 72), _F32)],
    )(x, batch3, gateW, gateb, outW, outb)


_MESH = plsc.VectorSubcoreMesh(core_axis_name="c", subcore_axis_name="s",
                               num_cores=2, num_subcores=16)


def _edge_pass_body(xh0, xh1, eh0, eh1, sidx, didx, out0, out1,
                    sG, dG, xrA, erA, xrB, erB, aggr, gsemA, gsemB):
    cid = lax.axis_index("c")
    sid = lax.axis_index("s")
    base = sid * ROWS_PER_TILE
    ebase = lax.rem(sid, 8) * EPT

    def zrow(i, _):
        erA[i, pl.ds(0, 16)] = jnp.zeros((16,), _F32)
        erA[i, pl.ds(16, 16)] = jnp.zeros((16,), _F32)
        return _
    lax.fori_loop(0, CHUNK, zrow, None)

    def zcp(c, _):
        pltpu.sync_copy(erA, aggr.at[pl.ds(base + c * CHUNK, CHUNK)])
        return _
    lax.fori_loop(0, ZCOPIES, zcp, None)
    plsc.subcore_barrier()

    def issue(curS, cloc, cglob, xr, er, sem):
        srow = curS.at[cloc]
        eh_rows = pl.ds(ebase + cglob * CHUNK, CHUNK)

        @pl.when(cid == 0)
        def _g0():
            pltpu.async_copy(xh0.at[srow], xr, sem)
            pltpu.async_copy(eh0.at[eh_rows], er, sem)

        @pl.when(cid == 1)
        def _g1():
            pltpu.async_copy(xh1.at[srow], xr, sem)
            pltpu.async_copy(eh1.at[eh_rows], er, sem)

    def drain(xr, er, sem):
        pltpu.make_async_copy(xh0.at[pl.ds(0, CHUNK)], xr, sem).wait()
        pltpu.make_async_copy(eh0.at[pl.ds(0, CHUNK)], er, sem).wait()

    def compute_scatter(curD, cloc, xr, er):
        def rows(i, _):
            s0 = pl.ds(0, 16)
            s1 = pl.ds(16, 16)
            xr[i, s0] = jnp.maximum(xr[i, s0] + er[i, s0], 0.0)
            xr[i, s1] = jnp.maximum(xr[i, s1] + er[i, s1], 0.0)
            return _
        lax.fori_loop(0, CHUNK, rows, None)
        pltpu.sync_copy(xr, aggr.at[curD.at[cloc]], add=True)

    def outer(g, _):
        gbase = g * GCH
        pltpu.sync_copy(sidx.at[sid, pl.ds(gbase, GCH)], sG)
        pltpu.sync_copy(didx.at[sid, pl.ds(gbase, GCH)], dG)
        issue(sG, 0, gbase, xrA, erA, gsemA)

        def pair(i, _):
            c0 = 2 * i
            issue(sG, c0 + 1, gbase + c0 + 1, xrB, erB, gsemB)
            drain(xrA, erA, gsemA)
            compute_scatter(dG, c0, xrA, erA)

            @pl.when(i < PAIRS - 1)
            def _next_even():
                issue(sG, c0 + 2, gbase + c0 + 2, xrA, erA, gsemA)

            drain(xrB, erB, gsemB)
            compute_scatter(dG, c0 + 1, xrB, erB)
            return _
        lax.fori_loop(0, PAIRS, pair, None)
        return _
    lax.fori_loop(0, NGROUPS, outer, None)
    plsc.subcore_barrier()

    @pl.when(cid == 0)
    def _w0():
        pltpu.sync_copy(aggr.at[pl.ds(base, ROWS_PER_TILE)],
                        out0.at[pl.ds(base, ROWS_PER_TILE)])

    @pl.when(cid == 1)
    def _w1():
        pltpu.sync_copy(aggr.at[pl.ds(base, ROWS_PER_TILE)],
                        out1.at[pl.ds(base, ROWS_PER_TILE)])


_edge_pass = pl.kernel(
    _edge_pass_body,
    out_type=[jax.ShapeDtypeStruct((NPAD, 32), _F32)] * 2,
    mesh=_MESH,
    compiler_params=pltpu.CompilerParams(use_tc_tiling_on_sc=False),
    scratch_types=[
        pltpu.VMEM((GCH, CHUNK), jnp.int32),
        pltpu.VMEM((GCH, CHUNK), jnp.int32),
        pltpu.VMEM((CHUNK, 32), _F32),
        pltpu.VMEM((CHUNK, 32), _F32),
        pltpu.VMEM((CHUNK, 32), _F32),
        pltpu.VMEM((CHUNK, 32), _F32),
        pltpu.VMEM_SHARED((NPAD, 32), _F32),
        pltpu.SemaphoreType.DMA,
        pltpu.SemaphoreType.DMA,
    ],
)


def _deg_body(didx, out0, out1, dbuf, ones, zbuf, degs):
    cid = lax.axis_index("c")
    sid = lax.axis_index("s")
    base = sid * ROWS_PER_TILE

    def fill(i, _):
        ones[i, pl.ds(0, 16)] = jnp.ones((16,), _F32)
        zbuf[i, pl.ds(0, 16)] = jnp.zeros((16,), _F32)
        return _
    lax.fori_loop(0, CHUNK, fill, None)

    def zcp(c, _):
        pltpu.sync_copy(zbuf, degs.at[pl.ds(base + c * CHUNK, CHUNK)])
        return _
    lax.fori_loop(0, ZCOPIES, zcp, None)
    plsc.subcore_barrier()

    def chunk(c, _):
        pltpu.sync_copy(didx.at[sid, cid * DEG_NCH + c], dbuf)
        pltpu.sync_copy(ones, degs.at[dbuf], add=True)
        return _
    lax.fori_loop(0, DEG_NCH, chunk, None)
    plsc.subcore_barrier()

    @pl.when(cid == 0)
    def _w0():
        pltpu.sync_copy(degs.at[pl.ds(base, ROWS_PER_TILE)],
                        out0.at[pl.ds(base, ROWS_PER_TILE)])

    @pl.when(cid == 1)
    def _w1():
        pltpu.sync_copy(degs.at[pl.ds(base, ROWS_PER_TILE)],
                        out1.at[pl.ds(base, ROWS_PER_TILE)])


_deg = pl.kernel(
    _deg_body,
    out_type=[jax.ShapeDtypeStruct((NPAD, 16), _F32)] * 2,
    mesh=_MESH,
    compiler_params=pltpu.CompilerParams(use_tc_tiling_on_sc=False),
    scratch_types=[
        pltpu.VMEM((CHUNK,), jnp.int32),
        pltpu.VMEM((CHUNK, 16), _F32),
        pltpu.VMEM((CHUNK, 16), _F32),
        pltpu.VMEM_SHARED((NPAD, 16), _F32),
    ],
)


def _tile_pack(a, pad_val):
    a = a.reshape(TILES, EPT)
    a = jnp.pad(a, ((0, 0), (0, NCH * CHUNK - EPT)), constant_values=pad_val)
    return a.reshape(TILES, NCH, CHUNK)


def kernel(node_features, edge_index, edge_features, batch, params):
    p = params
    r1 = lambda v: v.reshape(1, -1)

    x, xh0, xh1 = _node_mlp(node_features, p['nW1'], r1(p['nb1']),
                            p['nW2'], r1(p['nb2']), p['g1mW1'][:64])
    eh10, eh11, eh20, eh21 = _edge_mlp(
        edge_features, p['eW1'], r1(p['eb1']), p['eW2'], r1(p['eb2']),
        p['g1mW1'][64:], r1(p['g1mb1']), p['g2mW1'][64:], r1(p['g2mb1']))

    srcd = jnp.concatenate([edge_index[0], edge_index[1]])
    dstd = jnp.concatenate([edge_index[1], edge_index[0]])
    sidx = _tile_pack(srcd, 0)
    didx = _tile_pack(dstd, N)

    deg0, deg1 = _deg(didx)

    a0, a1 = _edge_pass(xh0, xh1, eh10, eh11, sidx, didx)
    x, xh0, xh1 = _update(a0, a1, deg0, deg1, x, p['g1mW2'], r1(p['g1mb2']),
                          p['g1uW'][:64], p['g1uW'][64:], r1(p['g1ub']),
                          nW1x=p['g2mW1'][:64])

    a0, a1 = _edge_pass(xh0, xh1, eh20, eh21, sidx, didx)
    x = _update(a0, a1, deg0, deg1, x, p['g2mW2'], r1(p['g2mb2']),
                p['g2uW'][:64], p['g2uW'][64:], r1(p['g2ub']))

    batch3 = batch.reshape(NBLK, RB, 1)
    return _readout(x, batch3, p['gateW'], r1(p['gateb']),
                    p['outW'], r1(p['outb']))


# speedup vs baseline: 7.6984x; 1.5860x over previous
import functools

import jax
import jax.numpy as jnp
from jax import lax
from jax.experimental import pallas as pl
from jax.experimental.pallas import tpu as pltpu
from jax.experimental.pallas import tpu_sc as plsc

N = 50000
E = 800000
ED = 2 * E
NG = 64

NPAD = 51200
TILES = 16
ROWS_PER_TILE = NPAD // TILES
ZCOPIES = ROWS_PER_TILE // 128
CHUNK = 128
EPT = ED // TILES
NCH = -(-EPT // CHUNK)
DEG_NCH = NCH // 2
GCH = 34
NGROUPS = NCH // GCH
PAIRS = GCH // 2
E_PAD = NCH * CHUNK + 7 * EPT
E_PAD4 = E_PAD // 4
EPT4 = EPT // 4

RB = 2000
NBLK = N // RB
EB = 4000
NEBLK = E // EB

_F32 = jnp.float32


def _node_mlp_body(nf, nW1, nb1, nW2, nb2, mW1x, x_out, xh0_out, xh1_out):
    a = jnp.maximum(jnp.dot(nf[...], nW1[...], preferred_element_type=_F32)
                    + nb1[...], 0.0)
    x = jnp.maximum(jnp.dot(a, nW2[...], preferred_element_type=_F32)
                    + nb2[...], 0.0)
    xh = jnp.dot(x, mW1x[...], preferred_element_type=_F32)
    x_out[...] = x
    xh0_out[...] = xh[:, :32]
    xh1_out[...] = xh[:, 32:]


def _node_mlp(nf, nW1, nb1, nW2, nb2, mW1x):
    full = lambda s: pl.BlockSpec(s, lambda i: (0, 0))
    row = lambda c: pl.BlockSpec((RB, c), lambda i: (i, 0))
    return pl.pallas_call(
        _node_mlp_body,
        grid=(NBLK,),
        in_specs=[row(128), full((128, 128)), full((1, 128)),
                  full((128, 64)), full((1, 64)), full((64, 64))],
        out_specs=[row(64), row(32), row(32)],
        out_shape=[jax.ShapeDtypeStruct((N, 64), _F32),
                   jax.ShapeDtypeStruct((N, 32), _F32),
                   jax.ShapeDtypeStruct((N, 32), _F32)],
    )(nf, nW1, nb1, nW2, nb2, mW1x)


def _edge_mlp_body(ef4, eW1, eb1, eW2, eb2, m10, m1b0, m11, m1b1,
                   m20, m2b0, m21, m2b1, o10, o11, o20, o21):
    a = jnp.maximum(jnp.dot(ef4[...], eW1[...], preferred_element_type=_F32)
                    + eb1[...], 0.0)
    ea = jnp.maximum(jnp.dot(a, eW2[...], preferred_element_type=_F32)
                     + eb2[...], 0.0)
    o10[...] = jnp.dot(ea, m10[...], preferred_element_type=_F32) + m1b0[...]
    o11[...] = jnp.dot(ea, m11[...], preferred_element_type=_F32) + m1b1[...]
    o20[...] = jnp.dot(ea, m20[...], preferred_element_type=_F32) + m2b0[...]
    o21[...] = jnp.dot(ea, m21[...], preferred_element_type=_F32) + m2b1[...]


def _edge_mlp(ef4, eW1, eb1, eW2, eb2, m10, m1b0, m11, m1b1,
              m20, m2b0, m21, m2b1):
    full = lambda s: pl.BlockSpec(s, lambda i: (0, 0))
    row = lambda c: pl.BlockSpec((EB // 4, c), lambda i: (i, 0))
    return pl.pallas_call(
        _edge_mlp_body,
        grid=(NEBLK,),
        in_specs=[row(64), full((64, 128)), full((1, 128)),
                  full((128, 64)), full((1, 64)),
                  full((64, 128)), full((1, 128)),
                  full((64, 128)), full((1, 128)),
                  full((64, 128)), full((1, 128)),
                  full((64, 128)), full((1, 128))],
        out_specs=[row(128), row(128), row(128), row(128)],
        out_shape=[jax.ShapeDtypeStruct((E_PAD4, 128), _F32)] * 4,
    )(ef4, eW1, eb1, eW2, eb2, m10, m1b0, m11, m1b1, m20, m2b0, m21, m2b1)


def _make_update_body(with_xh):
    def body(a0, a1, d0, d1, x, mW2, mb2, uWa, uWx, ub, *rest):
        if with_xh:
            (nW1x, xn_out, xh0_out, xh1_out) = rest
        else:
            (xn_out,) = rest
        s = jnp.concatenate([a0[...], a1[...]], axis=1)
        deg = d0[:, :1] + d1[:, :1]
        aggr = jnp.dot(s, mW2[...], preferred_element_type=_F32) + deg * mb2[...]
        xn = jnp.maximum(jnp.dot(aggr, uWa[...], preferred_element_type=_F32)
                         + jnp.dot(x[...], uWx[...], preferred_element_type=_F32)
                         + ub[...], 0.0)
        xn_out[...] = xn
        if with_xh:
            xh = jnp.dot(xn, nW1x[...], preferred_element_type=_F32)
            xh0_out[...] = xh[:, :32]
            xh1_out[...] = xh[:, 32:]
    return body


def _update(a0, a1, d0, d1, x, mW2, mb2, uWa, uWx, ub, nW1x=None):
    with_xh = nW1x is not None
    full = lambda s: pl.BlockSpec(s, lambda i: (0, 0))
    row = lambda c: pl.BlockSpec((RB, c), lambda i: (i, 0))
    in_specs = [row(32), row(32), row(16), row(16), row(64),
                full((64, 64)), full((1, 64)), full((64, 64)),
                full((64, 64)), full((1, 64))]
    args = [a0, a1, d0, d1, x, mW2, mb2, uWa, uWx, ub]
    out_specs = [row(64)]
    out_shape = [jax.ShapeDtypeStruct((N, 64), _F32)]
    if with_xh:
        in_specs.append(full((64, 64)))
        args.append(nW1x)
        out_specs += [row(32), row(32)]
        out_shape += [jax.ShapeDtypeStruct((N, 32), _F32)] * 2
    res = pl.pallas_call(
        _make_update_body(with_xh),
        grid=(NBLK,),
        in_specs=in_specs,
        out_specs=out_specs,
        out_shape=out_shape,
    )(*args)
    return res if with_xh else res[0]


def _readout_body(x, b, gateW, gateb, outW, outb, out, gmax_s, num_s):
    ph = pl.program_id(0)
    i = pl.program_id(1)

    @pl.when((ph == 0) & (i == 0))
    def _init():
        gmax_s[...] = jnp.full((1, NG), -1e30, _F32)
        num_s[...] = jnp.zeros((NG, 72), _F32)

    bcol = b[0]
    seg = lax.broadcasted_iota(jnp.int32, (1, NG), 1)
    mask = bcol == seg
    gate = jnp.dot(x[...], gateW[...], preferred_element_type=_F32) + gateb[...]

    @pl.when(ph == 0)
    def _maxpass():
        contrib = jnp.where(mask, gate, -1e30)
        gmax_s[...] = jnp.maximum(gmax_s[...],
                                  jnp.max(contrib, axis=0, keepdims=True))

    @pl.when(ph == 1)
    def _sumpass():
        gm = gmax_s[...]
        gm = jnp.where(gm > -1e29, gm, 0.0)
        gsel = jnp.sum(jnp.where(mask, gm, 0.0), axis=1, keepdims=True)
        e = jnp.exp(gate - gsel)
        em = jnp.where(mask, e, 0.0)
        xext = jnp.concatenate([x[...], jnp.ones((RB, 8), _F32)], axis=1)
        num_s[...] += lax.dot_general(em, xext,
                                      (((0,), (0,)), ((), ())),
                                      preferred_element_type=_F32)

    @pl.when((ph == 2) & (i == 0))
    def _finish():
        nv = num_s[...]
        den = nv[:, 64:65]
        ro = nv[:, :64] / (den + 1e-16)
        out[...] = jnp.dot(ro, outW[...], preferred_element_type=_F32) + outb[...]


def _readout(x, batch3, gateW, gateb, outW, outb):
    full = lambda s: pl.BlockSpec(s, lambda ph, i: (0, 0))
    return pl.pallas_call(
        _readout_body,
        grid=(3, NBLK),
        in_specs=[pl.BlockSpec((RB, 64), lambda ph, i: (i, 0)),
                  pl.BlockSpec((1, RB, 1), lambda ph, i: (i, 0, 0)),
                  full((64, 1)), full((1, 1)), full((64, 32)), full((1, 32))],
        out_specs=pl.BlockSpec((NG, 32), lambda ph, i: (0, 0)),
        out_shape=jax.ShapeDtypeStruct((NG, 32), _F32),
        scratch_shapes=[pltpu.VMEM((1, NG), _F32), pltpu.VMEM((NG, 72), _F32)],
    )(x, batch3, gateW, gateb, outW, outb)


_MESH = plsc.VectorSubcoreMesh(core_axis_name="c", subcore_axis_name="s",
                               num_cores=2, num_subcores=16)


def _edge_pass_body(xh0, xh1, eh0, eh1, sidx, didx, out0, out1,
                    sG, dG, xrA, erA, xrB, erB, aggr, gsemA, gsemB):
    cid = lax.axis_index("c")
    sid = lax.axis_index("s")
    base = sid * ROWS_PER_TILE
    ebase4 = lax.rem(sid, 8) * EPT4

    def zrow(i, _):
        xrA[i, pl.ds(0, 16)] = jnp.zeros((16,), _F32)
        xrA[i, pl.ds(16, 16)] = jnp.zeros((16,), _F32)
        return _
    lax.fori_loop(0, CHUNK, zrow, None)

    def zcp(c, _):
        pltpu.sync_copy(xrA, aggr.at[pl.ds(base + c * CHUNK, CHUNK)])
        return _
    lax.fori_loop(0, ZCOPIES, zcp, None)
    plsc.subcore_barrier()

    def issue(curS, cloc, cglob, xr, er, sem):
        srow = curS.at[cloc]
        eh_rows = pl.ds(ebase4 + cglob * 32, 32)

        @pl.when(cid == 0)
        def _g0():
            pltpu.async_copy(xh0.at[srow], xr, sem)
            pltpu.async_copy(eh0.at[eh_rows], er, sem)

        @pl.when(cid == 1)
        def _g1():
            pltpu.async_copy(xh1.at[srow], xr, sem)
            pltpu.async_copy(eh1.at[eh_rows], er, sem)

    def drain(xr, er, sem):
        pltpu.make_async_copy(xh0.at[pl.ds(0, CHUNK)], xr, sem).wait()
        pltpu.make_async_copy(eh0.at[pl.ds(0, 32)], er, sem).wait()

    def compute_scatter(curD, cloc, xr, er):
        def rows(r, _):
            for sub in range(4):
                for h in range(2):
                    xs = pl.ds(16 * h, 16)
                    es = pl.ds(32 * sub + 16 * h, 16)
                    xr[4 * r + sub, xs] = jnp.maximum(
                        xr[4 * r + sub, xs] + er[r, es], 0.0)
            return _
        lax.fori_loop(0, 32, rows, None)
        pltpu.sync_copy(xr, aggr.at[curD.at[cloc]], add=True)

    def outer(g, _):
        gbase = g * GCH
        pltpu.sync_copy(sidx.at[sid, pl.ds(gbase, GCH)], sG)
        pltpu.sync_copy(didx.at[sid, pl.ds(gbase, GCH)], dG)
        issue(sG, 0, gbase, xrA, erA, gsemA)

        def pair(i, _):
            c0 = 2 * i
            issue(sG, c0 + 1, gbase + c0 + 1, xrB, erB, gsemB)
            drain(xrA, erA, gsemA)
            compute_scatter(dG, c0, xrA, erA)

            @pl.when(i < PAIRS - 1)
            def _next_even():
                issue(sG, c0 + 2, gbase + c0 + 2, xrA, erA, gsemA)

            drain(xrB, erB, gsemB)
            compute_scatter(dG, c0 + 1, xrB, erB)
            return _
        lax.fori_loop(0, PAIRS, pair, None)
        return _
    lax.fori_loop(0, NGROUPS, outer, None)
    plsc.subcore_barrier()

    @pl.when(cid == 0)
    def _w0():
        pltpu.sync_copy(aggr.at[pl.ds(base, ROWS_PER_TILE)],
                        out0.at[pl.ds(base, ROWS_PER_TILE)])

    @pl.when(cid == 1)
    def _w1():
        pltpu.sync_copy(aggr.at[pl.ds(base, ROWS_PER_TILE)],
                        out1.at[pl.ds(base, ROWS_PER_TILE)])


_edge_pass = pl.kernel(
    _edge_pass_body,
    out_type=[jax.ShapeDtypeStruct((NPAD, 32), _F32)] * 2,
    mesh=_MESH,
    compiler_params=pltpu.CompilerParams(use_tc_tiling_on_sc=False),
    scratch_types=[
        pltpu.VMEM((GCH, CHUNK), jnp.int32),
        pltpu.VMEM((GCH, CHUNK), jnp.int32),
        pltpu.VMEM((CHUNK, 32), _F32),
        pltpu.VMEM((32, 128), _F32),
        pltpu.VMEM((CHUNK, 32), _F32),
        pltpu.VMEM((32, 128), _F32),
        pltpu.VMEM_SHARED((NPAD, 32), _F32),
        pltpu.SemaphoreType.DMA,
        pltpu.SemaphoreType.DMA,
    ],
)


def _deg_body(didx, out0, out1, dbuf, ones, zbuf, degs):
    cid = lax.axis_index("c")
    sid = lax.axis_index("s")
    base = sid * ROWS_PER_TILE

    def fill(i, _):
        ones[i, pl.ds(0, 16)] = jnp.ones((16,), _F32)
        zbuf[i, pl.ds(0, 16)] = jnp.zeros((16,), _F32)
        return _
    lax.fori_loop(0, CHUNK, fill, None)

    def zcp(c, _):
        pltpu.sync_copy(zbuf, degs.at[pl.ds(base + c * CHUNK, CHUNK)])
        return _
    lax.fori_loop(0, ZCOPIES, zcp, None)
    plsc.subcore_barrier()

    def chunk(c, _):
        pltpu.sync_copy(didx.at[sid, cid * DEG_NCH + c], dbuf)
        pltpu.sync_copy(ones, degs.at[dbuf], add=True)
        return _
    lax.fori_loop(0, DEG_NCH, chunk, None)
    plsc.subcore_barrier()

    @pl.when(cid == 0)
    def _w0():
        pltpu.sync_copy(degs.at[pl.ds(base, ROWS_PER_TILE)],
                        out0.at[pl.ds(base, ROWS_PER_TILE)])

    @pl.when(cid == 1)
    def _w1():
        pltpu.sync_copy(degs.at[pl.ds(base, ROWS_PER_TILE)],
                        out1.at[pl.ds(base, ROWS_PER_TILE)])


_deg = pl.kernel(
    _deg_body,
    out_type=[jax.ShapeDtypeStruct((NPAD, 16), _F32)] * 2,
    mesh=_MESH,
    compiler_params=pltpu.CompilerParams(use_tc_tiling_on_sc=False),
    scratch_types=[
        pltpu.VMEM((CHUNK,), jnp.int32),
        pltpu.VMEM((CHUNK, 16), _F32),
        pltpu.VMEM((CHUNK, 16), _F32),
        pltpu.VMEM_SHARED((NPAD, 16), _F32),
    ],
)


def _tile_pack(a, pad_val):
    a = a.reshape(TILES, EPT)
    a = jnp.pad(a, ((0, 0), (0, NCH * CHUNK - EPT)), constant_values=pad_val)
    return a.reshape(TILES, NCH, CHUNK)


def kernel(node_features, edge_index, edge_features, batch, params):
    p = params
    r1 = lambda v: v.reshape(1, -1)

    x, xh0, xh1 = _node_mlp(node_features, p['nW1'], r1(p['nb1']),
                            p['nW2'], r1(p['nb2']), p['g1mW1'][:64])

    eye4 = jnp.eye(4, dtype=_F32)
    bd = lambda w: jnp.kron(eye4, w)
    t4 = lambda b: r1(jnp.tile(b, 4))
    m1e, m2e = p['g1mW1'][64:], p['g2mW1'][64:]
    eh10, eh11, eh20, eh21 = _edge_mlp(
        edge_features.reshape(E // 4, 64),
        bd(p['eW1']), t4(p['eb1']), bd(p['eW2']), t4(p['eb2']),
        bd(m1e[:, :32]), t4(p['g1mb1'][:32]), bd(m1e[:, 32:]), t4(p['g1mb1'][32:]),
        bd(m2e[:, :32]), t4(p['g2mb1'][:32]), bd(m2e[:, 32:]), t4(p['g2mb1'][32:]))

    srcd = jnp.concatenate([edge_index[0], edge_index[1]])
    dstd = jnp.concatenate([edge_index[1], edge_index[0]])
    sidx = _tile_pack(srcd, 0)
    didx = _tile_pack(dstd, N)

    deg0, deg1 = _deg(didx)

    a0, a1 = _edge_pass(xh0, xh1, eh10, eh11, sidx, didx)
    x, xh0, xh1 = _update(a0, a1, deg0, deg1, x, p['g1mW2'], r1(p['g1mb2']),
                          p['g1uW'][:64], p['g1uW'][64:], r1(p['g1ub']),
                          nW1x=p['g2mW1'][:64])

    a0, a1 = _edge_pass(xh0, xh1, eh20, eh21, sidx, didx)
    x = _update(a0, a1, deg0, deg1, x, p['g2mW2'], r1(p['g2mb2']),
                p['g2uW'][:64], p['g2uW'][64:], r1(p['g2ub']))

    batch3 = batch.reshape(NBLK, RB, 1)
    return _readout(x, batch3, p['gateW'], r1(p['gateb']),
                    p['outW'], r1(p['outb']))
